# baseline (speedup 1.0000x reference)
import functools

import numpy as np

import jax
import jax.numpy as jnp
from jax import lax
from jax.experimental import pallas as pl
from jax.experimental.pallas import tpu as pltpu
from jax.experimental.pallas import tpu_sc as plsc

_B = 128
_NC = 2
_NS = 16
_NW = _NC * _NS
_CP = pltpu.CompilerParams(use_tc_tiling_on_sc=False, needs_layout_passes=False)


def _iota16():
    return lax.iota(jnp.int32, 16)


def _splat_i(x):
    return jnp.full((16,), x, jnp.int32)


@functools.cache
def _phase_a(E_real, E_pad, F, H, N_pad):
    mesh = plsc.VectorSubcoreMesh(core_axis_name="c", subcore_axis_name="s",
                                  num_cores=_NC, num_subcores=_NS)
    nblk = E_pad // (_NW * _B)
    drpt = N_pad // _NS
    out_type = tuple([jax.ShapeDtypeStruct((_NC * N_pad, 8), jnp.float32)] +
                     [jax.ShapeDtypeStruct((E_pad, 64), jnp.float32)
                      for _ in range(H)])
    scratch = ([
        pltpu.VMEM((_B,), jnp.int32),
        pltpu.VMEM((_B,), jnp.int32),
        pltpu.VMEM((_B, F), jnp.float32),
        pltpu.VMEM((_B, F), jnp.float32),
        pltpu.VMEM((_B, 8), jnp.float32),
        pltpu.VMEM((F,), jnp.float32),
        pltpu.VMEM((_B, 8), jnp.float32),
        pltpu.VMEM_SHARED((N_pad, 8), jnp.float32),
    ] + [pltpu.VMEM((_B, 64), jnp.float32) for _ in range(H)]
      + [pltpu.SemaphoreType.DMA, pltpu.SemaphoreType.DMA])

    def body(*refs):
        (xl, xr, att, srcp, dstp, den2_o) = refs[:6]
        msg_o = refs[6:6 + H]
        (srcv, dstv, rows_l, rows_r, denblk, attv, zbuf, dacc) = refs[6 + H:14 + H]
        msgb = refs[14 + H:14 + 2 * H]
        sem1, sem2 = refs[14 + 2 * H:]
        c = lax.axis_index("c")
        s = lax.axis_index("s")
        wid = s * _NC + c
        it = _iota16()
        zf = jnp.zeros((16,), jnp.float32)
        for col in range(3):
            for r in range(_B // 16):
                plsc.store_scatter(zbuf, [r * 16 + it, _splat_i(col)], zf)
                plsc.store_scatter(denblk, [r * 16 + it, _splat_i(col)], zf)
        def zden(k, _):
            pltpu.sync_copy(zbuf, dacc.at[pl.ds(s * drpt + k * _B, _B)])
            return 0
        lax.fori_loop(0, drpt // _B, zden, 0)
        pltpu.sync_copy(att, attv)
        plsc.subcore_barrier()

        def block(i, _):
            base = (wid * nblk + i) * _B
            pltpu.sync_copy(srcp.at[pl.ds(base, _B)], srcv)
            pltpu.sync_copy(dstp.at[pl.ds(base, _B)], dstv)
            cp1 = pltpu.async_copy(xl.at[srcv], rows_l, sem1)
            cp2 = pltpu.async_copy(xr.at[dstv], rows_r, sem2)
            cp1.wait()
            cp2.wait()
            for g in range(_B // 16):
                rowi = g * 16 + it
                eids = base + rowi
                mask = eids < E_real

                def dbody(d, acc):
                    ds_ = _splat_i(d)
                    vl = plsc.load_gather(rows_l, [rowi, ds_])
                    vr = plsc.load_gather(rows_r, [rowi, ds_])
                    sm = vl + vr
                    e = jnp.maximum(sm, 0.2 * sm)
                    ad = plsc.load_gather(attv, [ds_])
                    return acc + e * ad

                for h in range(H):
                    acc = lax.fori_loop(h * 64, (h + 1) * 64, dbody, zf)
                    ex = jnp.where(mask, jnp.exp(acc), 0.0)
                    plsc.store_scatter(denblk, [rowi, _splat_i(h)], ex)

                    def sbody(d, _):
                        ds_ = _splat_i(d)
                        v = plsc.load_gather(rows_l, [rowi, ds_])
                        plsc.store_scatter(msgb[h], [rowi, ds_ - h * 64], v * ex)
                        return 0
                    lax.fori_loop(h * 64, (h + 1) * 64, sbody, 0)
                cnt = jnp.where(mask, 1.0, 0.0)
                plsc.store_scatter(denblk, [rowi, _splat_i(2)], cnt)
            for h in range(H):
                pltpu.sync_copy(msgb[h], msg_o[h].at[pl.ds(base, _B)])
            pltpu.sync_copy(denblk, dacc.at[dstv], add=True)
            return 0

        lax.fori_loop(0, nblk, block, 0)
        plsc.subcore_barrier()

        def wout(k, _):
            off = s * drpt + k * _B
            pltpu.sync_copy(dacc.at[pl.ds(off, _B)],
                            den2_o.at[pl.ds(c * N_pad + off, _B)])
            return 0
        lax.fori_loop(0, drpt // _B, wout, 0)

    return pl.kernel(body, out_type=out_type, mesh=mesh, compiler_params=_CP,
                     scratch_types=scratch, name=f"gat_a_{E_pad}_{F}_{H}")


@functools.cache
def _phase_b(E_pad, H, N_pad):
    mesh = plsc.VectorSubcoreMesh(core_axis_name="c", subcore_axis_name="s",
                                  num_cores=_NC, num_subcores=_NS)
    qsz = N_pad // 4
    arpt = qsz // _NS
    nbt = E_pad // (_NS * _B)
    out_type = tuple(jax.ShapeDtypeStruct((N_pad, 64), jnp.float32)
                     for _ in range(H))
    scratch = [
        pltpu.VMEM((_B,), jnp.int32),
        pltpu.VMEM((_B,), jnp.int32),
        pltpu.VMEM((_B, 64), jnp.float32),
        pltpu.VMEM((64, 64), jnp.float32),
        pltpu.VMEM_SHARED((qsz + 8, 64), jnp.float32),
        pltpu.SemaphoreType.DMA,
    ]

    def body(*refs):
        dstp = refs[0]
        msgs = refs[1:1 + H]
        outs = refs[1 + H:1 + 2 * H]
        (dstv, idxb, rows, zbuf, acc, sem) = refs[1 + 2 * H:]
        c = lax.axis_index("c")
        s = lax.axis_index("s")
        it = _iota16()
        zf = jnp.zeros((16,), jnp.float32)

        def zz(i, _):
            fl = i * 16 + it
            plsc.store_scatter(zbuf, [fl // 64, fl % 64], zf)
            return 0
        lax.fori_loop(0, 64 * 64 // 16, zz, 0)

        def zacc(k, _):
            pltpu.sync_copy(zbuf, acc.at[pl.ds(s * arpt + k * 64, 64)])
            return 0

        for h in range(H):
            for qq in range(2):
                q = c * 2 + qq
                qbase = q * qsz
                lax.fori_loop(0, arpt // 64, zacc, 0)
                plsc.subcore_barrier()

                def block(b, _):
                    base = (s * nbt + b) * _B
                    pltpu.sync_copy(dstp.at[pl.ds(base, _B)], dstv)
                    cp = pltpu.async_copy(msgs[h].at[pl.ds(base, _B)], rows, sem)
                    for g in range(_B // 16):
                        d16 = dstv[pl.ds(g * 16, 16)]
                        local = d16 - qbase
                        msk = (local >= 0) & (local < qsz)
                        idxb[pl.ds(g * 16, 16)] = jnp.where(msk, local, qsz)
                    cp.wait()
                    pltpu.sync_copy(rows, acc.at[idxb], add=True)
                    return 0

                lax.fori_loop(0, nbt, block, 0)
                plsc.subcore_barrier()

                def wout(k, _):
                    off = s * arpt + k * 64
                    pltpu.sync_copy(acc.at[pl.ds(off, 64)],
                                    outs[h].at[pl.ds(qbase + off, 64)])
                    return 0
                lax.fori_loop(0, arpt // 64, wout, 0)
                if h + 1 < H or qq == 0:
                    plsc.subcore_barrier()

    return pl.kernel(body, out_type=out_type, mesh=mesh, compiler_params=_CP,
                     scratch_types=scratch, name=f"gat_b_{E_pad}_{H}")


def _ceil_to(x, m):
    return (x + m - 1) // m * m


_BN = 1024


def _tc_grid(n):
    return (n + _BN - 1) // _BN


@functools.cache
def _tc_mm(N, K, M, relu):
    def body(x_ref, w_ref, b_ref, o_ref):
        o = jnp.dot(x_ref[...], w_ref[...],
                    preferred_element_type=jnp.float32) + b_ref[...]
        o_ref[...] = jnp.maximum(o, 0.0) if relu else o

    return pl.pallas_call(
        body,
        grid=(_tc_grid(N),),
        in_specs=[pl.BlockSpec((_BN, K), lambda i: (i, 0)),
                  pl.BlockSpec((K, M), lambda i: (0, 0)),
                  pl.BlockSpec((1, M), lambda i: (0, 0))],
        out_specs=pl.BlockSpec((_BN, M), lambda i: (i, 0)),
        out_shape=jax.ShapeDtypeStruct((N, M), jnp.float32),
        name=f"mm_{N}_{K}_{M}")


def _proj(x, w, b, relu=False):
    return _tc_mm(x.shape[0], x.shape[1], w.shape[1], relu)(x, w, b.reshape(1, -1))


@functools.cache
def _tc_cnn(N):
    def body(x_ref, k_ref, b_ref, s_ref, o_ref):
        h = x_ref[...]
        for i in range(3):
            h = jnp.dot(h, k_ref[i], preferred_element_type=jnp.float32)
            h = jnp.maximum(h + b_ref[i], 0.0)
        pool = jnp.dot(h, s_ref[0], preferred_element_type=jnp.float32)
        for k in range(1, 4):
            pool = jnp.maximum(
                pool, jnp.dot(h, s_ref[k], preferred_element_type=jnp.float32))
        o_ref[...] = pool

    return pl.pallas_call(
        body,
        grid=(_tc_grid(N),),
        in_specs=[pl.BlockSpec((_BN, 125), lambda i: (i, 0)),
                  pl.BlockSpec((3, 125, 125), lambda i: (0, 0, 0)),
                  pl.BlockSpec((3, 1, 125), lambda i: (0, 0, 0)),
                  pl.BlockSpec((4, 125, 45), lambda i: (0, 0, 0))],
        out_specs=pl.BlockSpec((_BN, 45), lambda i: (i, 0)),
        out_shape=jax.ShapeDtypeStruct((N, 45), jnp.float32),
        name="cnn")


@functools.cache
def _pool_sel():
    S = np.zeros((4, 125, 45), np.float32)
    win = [(0,), (1, 2), (3, 4)]
    for c in range(5):
        for i in range(3):
            for j in range(3):
                co = c * 9 + i * 3 + j
                R, C = win[i], win[j]
                for k, (a, b) in enumerate(((0, 0), (0, 1), (1, 0), (1, 1))):
                    r = R[min(a, len(R) - 1)]
                    cc = C[min(b, len(C) - 1)]
                    S[k, c * 25 + r * 5 + cc, co] = 1.0
    return S


def _cnn(x, p):
    mats, biases = [], []
    for i in (1, 2, 3):
        w = p['conv%d_w' % i]
        g, b2, m, v = (p['bn%d_g' % i], p['bn%d_b' % i],
                       p['bn%d_m' % i], p['bn%d_v' % i])
        scale = g / jnp.sqrt(v + 1e-5)
        shift = b2 - m * scale
        PI, PO, CC, DI, DJ = [], [], [], [], []
        for c in range(5):
            for ri in range(5):
                for cj in range(5):
                    for di in range(3):
                        for dj in range(3):
                            ii, jj = ri - di + 1, cj - dj + 1
                            if 0 <= ii < 5 and 0 <= jj < 5:
                                PI.append(c * 25 + ri * 5 + cj)
                                PO.append(c * 25 + ii * 5 + jj)
                                CC.append(c)
                                DI.append(di)
                                DJ.append(dj)
        K = jnp.zeros((125, 125), jnp.float32).at[
            np.array(PI), np.array(PO)].set(w[np.array(CC), 0, np.array(DI), np.array(DJ)])
        colscale = jnp.repeat(scale, 25)
        mats.append(K * colscale[None, :])
        biases.append((jnp.repeat(p['conv%d_b' % i], 25) * colscale
                       + jnp.repeat(shift, 25)).reshape(1, 125))
    kmat = jnp.stack(mats)
    bmat = jnp.stack(biases)
    S = jnp.asarray(_pool_sel())
    x2 = x.reshape(x.shape[0], 125)
    return _tc_cnn(x2.shape[0])(x2, kmat, bmat, S)


@functools.cache
def _tc_finish(N, H, relu, bn):
    F = H * 64

    def body(*refs):
        s_refs = refs[:H]
        d0_ref, d1_ref, bias_ref, sc_ref, sh_ref, o_ref = refs[H:]
        cnt = jnp.maximum(d0_ref[:, 2:3] + d1_ref[:, 2:3], 1.0)
        cols = []
        for h in range(H):
            den = d0_ref[:, h:h + 1] + d1_ref[:, h:h + 1]
            cols.append(s_refs[h][...] / (den * cnt + 1e-30))
        o = jnp.concatenate(cols, axis=1) + bias_ref[...]
        if bn:
            o = o * sc_ref[...] + sh_ref[...]
        o_ref[...] = jnp.maximum(o, 0.0) if relu else o

    return pl.pallas_call(
        body,
        grid=(_tc_grid(N),),
        in_specs=[pl.BlockSpec((_BN, 64), lambda i: (i, 0)) for _ in range(H)]
        + [pl.BlockSpec((_BN, 8), lambda i: (i, 0)),
           pl.BlockSpec((_BN, 8), lambda i: (i, 0)),
           pl.BlockSpec((1, F), lambda i: (0, 0)),
           pl.BlockSpec((1, F), lambda i: (0, 0)),
           pl.BlockSpec((1, F), lambda i: (0, 0))],
        out_specs=pl.BlockSpec((_BN, F), lambda i: (i, 0)),
        out_shape=jax.ShapeDtypeStruct((N, F), jnp.float32),
        name=f"finish_{N}_{H}")


@functools.cache
def _tc_mlp(N):
    def body(x_ref, w1, b1, w2, b2, w3, b3, o_ref):
        h = jnp.maximum(jnp.dot(x_ref[...], w1[...],
                                preferred_element_type=jnp.float32) + b1[...], 0.0)
        h = jnp.maximum(jnp.dot(h, w2[...],
                                preferred_element_type=jnp.float32) + b2[...], 0.0)
        o_ref[...] = jnp.dot(h, w3[...],
                             preferred_element_type=jnp.float32) + b3[...]

    return pl.pallas_call(
        body,
        grid=(_tc_grid(N),),
        in_specs=[pl.BlockSpec((_BN, 64), lambda i: (i, 0)),
                  pl.BlockSpec((64, 64), lambda i: (0, 0)),
                  pl.BlockSpec((1, 64), lambda i: (0, 0)),
                  pl.BlockSpec((64, 32), lambda i: (0, 0)),
                  pl.BlockSpec((1, 32), lambda i: (0, 0)),
                  pl.BlockSpec((32, 1), lambda i: (0, 0)),
                  pl.BlockSpec((1, 1), lambda i: (0, 0))],
        out_specs=pl.BlockSpec((_BN, 1), lambda i: (i, 0)),
        out_shape=jax.ShapeDtypeStruct((N, 1), jnp.float32),
        name="mlp")


def _pad1(a, n):
    return jnp.concatenate([a, jnp.zeros((n - a.shape[0],), a.dtype)])


def _sc_gat(x_src, x_dst, ei, p, name, heads, num_dst, bn=None, relu=False):
    if x_src is x_dst:
        w2 = jnp.concatenate([p[name + '_Wl'], p[name + '_Wr']], axis=1)
        b2 = jnp.concatenate([p[name + '_bl'], p[name + '_br']])
        xlr = _proj(x_src, w2, b2)
        F = heads * 64
        xl, xr = xlr[:, :F], xlr[:, F:]
    else:
        xl = _proj(x_src, p[name + '_Wl'], p[name + '_bl'])
        xr = _proj(x_dst, p[name + '_Wr'], p[name + '_br'])
    F = heads * 64
    E = ei.shape[1]
    E_pad = _ceil_to(E, _NW * _B)
    N_pad = _ceil_to(num_dst, 4 * _NS * 64)
    srcp = _pad1(ei[0], E_pad)
    dstp = _pad1(ei[1], E_pad)
    att = p[name + '_att'].reshape(-1)
    res = _phase_a(E, E_pad, F, heads, N_pad)(xl, xr, att, srcp, dstp)
    den2, msgs = res[0], res[1:]
    den2 = den2.reshape(_NC, N_pad, 8)
    if bn is not None:
        g, b2_, m, v = bn
        scale = g / jnp.sqrt(v + 1e-5)
        shift = b2_ - m * scale
    else:
        scale = jnp.ones((F,), jnp.float32)
        shift = jnp.zeros((F,), jnp.float32)
    outs = _phase_b(E_pad, heads, N_pad)(dstp, *msgs)
    out = _tc_finish(N_pad, heads, relu, bn is not None)(
        *outs, den2[0], den2[1], p[name + '_bias'].reshape(1, -1),
        scale.reshape(1, -1), shift.reshape(1, -1))
    return out[:num_dst]


def kernel(x_low, x_9x, x_25x, x_high, z_std_high, ei_low_9x, ei_9x_25x, ei_25x_high, ei_high, params):
    p = params
    h = _cnn(x_low, p)
    h = _sc_gat(h, x_9x, ei_low_9x, p, 'd1', 1, x_9x.shape[0])
    h = _sc_gat(h, x_25x, ei_9x_25x, p, 'd2', 1, x_25x.shape[0])
    h = _sc_gat(h, x_high, ei_25x_high, p, 'd3', 1, x_high.shape[0])
    h = jnp.concatenate([z_std_high, h], axis=-1)
    n = h.shape[0]
    loops = jnp.arange(n, dtype=ei_high.dtype)
    ei = jnp.concatenate([ei_high, jnp.stack([loops, loops])], axis=1)
    for i in (1, 2, 3, 4):
        bn = (p['pbn%d_g' % i], p['pbn%d_b' % i], p['pbn%d_m' % i], p['pbn%d_v' % i])
        h = _sc_gat(h, h, ei, p, 'p%d' % i, 2, n, bn=bn, relu=True)
    h = _sc_gat(h, h, ei, p, 'p5', 1, n, relu=True)
    return _tc_mlp(n)(h, p['fc1_w'], p['fc1_b'].reshape(1, -1),
                      p['fc2_w'], p['fc2_b'].reshape(1, -1),
                      p['fc3_w'], p['fc3_b'].reshape(1, -1))


# speedup vs baseline: 1.0044x; 1.0044x over previous
import functools

import numpy as np

import jax
import jax.numpy as jnp
from jax import lax
from jax.experimental import pallas as pl
from jax.experimental.pallas import tpu as pltpu
from jax.experimental.pallas import tpu_sc as plsc

_B = 128
_NC = 2
_NS = 16
_NW = _NC * _NS
_CP = pltpu.CompilerParams(use_tc_tiling_on_sc=False, needs_layout_passes=False)


def _iota16():
    return lax.iota(jnp.int32, 16)


def _splat_i(x):
    return jnp.full((16,), x, jnp.int32)


@functools.cache
def _phase_a(E_real, E_pad, F, H, N_pad):
    mesh = plsc.VectorSubcoreMesh(core_axis_name="c", subcore_axis_name="s",
                                  num_cores=_NC, num_subcores=_NS)
    nblk = E_pad // (_NW * _B)
    drpt = N_pad // _NS
    out_type = tuple([jax.ShapeDtypeStruct((_NC * N_pad, 8), jnp.float32)] +
                     [jax.ShapeDtypeStruct((E_pad, 64), jnp.float32)
                      for _ in range(H)])
    scratch = ([
        pltpu.VMEM((_B,), jnp.int32),
        pltpu.VMEM((_B,), jnp.int32),
        pltpu.VMEM((_B, F), jnp.float32),
        pltpu.VMEM((_B, F), jnp.float32),
        pltpu.VMEM((_B, 8), jnp.float32),
        pltpu.VMEM((F,), jnp.float32),
        pltpu.VMEM((_B, 8), jnp.float32),
        pltpu.VMEM_SHARED((N_pad, 8), jnp.float32),
    ] + [pltpu.VMEM((_B, 64), jnp.float32) for _ in range(H)]
      + [pltpu.SemaphoreType.DMA, pltpu.SemaphoreType.DMA])

    def body(*refs):
        (xl, xr, att, srcp, dstp, den2_o) = refs[:6]
        msg_o = refs[6:6 + H]
        (srcv, dstv, rows_l, rows_r, denblk, attv, zbuf, dacc) = refs[6 + H:14 + H]
        msgb = refs[14 + H:14 + 2 * H]
        sem1, sem2 = refs[14 + 2 * H:]
        c = lax.axis_index("c")
        s = lax.axis_index("s")
        wid = s * _NC + c
        it = _iota16()
        zf = jnp.zeros((16,), jnp.float32)
        for col in range(3):
            for r in range(_B // 16):
                plsc.store_scatter(zbuf, [r * 16 + it, _splat_i(col)], zf)
                plsc.store_scatter(denblk, [r * 16 + it, _splat_i(col)], zf)
        def zden(k, _):
            pltpu.sync_copy(zbuf, dacc.at[pl.ds(s * drpt + k * _B, _B)])
            return 0
        lax.fori_loop(0, drpt // _B, zden, 0)
        pltpu.sync_copy(att, attv)
        plsc.subcore_barrier()

        def block(i, _):
            base = (wid * nblk + i) * _B
            pltpu.sync_copy(srcp.at[pl.ds(base, _B)], srcv)
            pltpu.sync_copy(dstp.at[pl.ds(base, _B)], dstv)
            cp1 = pltpu.async_copy(xl.at[srcv], rows_l, sem1)
            cp2 = pltpu.async_copy(xr.at[dstv], rows_r, sem2)
            cp1.wait()
            cp2.wait()
            for g in range(_B // 16):
                rowi = g * 16 + it
                eids = base + rowi
                mask = eids < E_real

                def dbody(k, acc):
                    d0 = k * 4
                    for j in range(4):
                        ds_ = _splat_i(d0 + j)
                        vl = plsc.load_gather(rows_l, [rowi, ds_])
                        vr = plsc.load_gather(rows_r, [rowi, ds_])
                        sm = vl + vr
                        e = jnp.maximum(sm, 0.2 * sm)
                        ad = plsc.load_gather(attv, [ds_])
                        acc = acc + e * ad
                    return acc

                for h in range(H):
                    acc = lax.fori_loop(h * 16, (h + 1) * 16, dbody, zf)
                    ex = jnp.where(mask, jnp.exp(acc), 0.0)
                    plsc.store_scatter(denblk, [rowi, _splat_i(h)], ex)

                    def sbody(k, _):
                        d0 = k * 4
                        for j in range(4):
                            ds_ = _splat_i(d0 + j)
                            v = plsc.load_gather(rows_l, [rowi, ds_])
                            plsc.store_scatter(msgb[h], [rowi, ds_ - h * 64],
                                               v * ex)
                        return 0
                    lax.fori_loop(h * 16, (h + 1) * 16, sbody, 0)
                cnt = jnp.where(mask, 1.0, 0.0)
                plsc.store_scatter(denblk, [rowi, _splat_i(2)], cnt)
            for h in range(H):
                pltpu.sync_copy(msgb[h], msg_o[h].at[pl.ds(base, _B)])
            pltpu.sync_copy(denblk, dacc.at[dstv], add=True)
            return 0

        lax.fori_loop(0, nblk, block, 0)
        plsc.subcore_barrier()

        def wout(k, _):
            off = s * drpt + k * _B
            pltpu.sync_copy(dacc.at[pl.ds(off, _B)],
                            den2_o.at[pl.ds(c * N_pad + off, _B)])
            return 0
        lax.fori_loop(0, drpt // _B, wout, 0)

    return pl.kernel(body, out_type=out_type, mesh=mesh, compiler_params=_CP,
                     scratch_types=scratch, name=f"gat_a_{E_pad}_{F}_{H}")


@functools.cache
def _phase_b(E_pad, H, N_pad):
    mesh = plsc.VectorSubcoreMesh(core_axis_name="c", subcore_axis_name="s",
                                  num_cores=_NC, num_subcores=_NS)
    qsz = N_pad // 4
    arpt = qsz // _NS
    nbt = E_pad // (_NS * _B)
    out_type = tuple(jax.ShapeDtypeStruct((N_pad, 64), jnp.float32)
                     for _ in range(H))
    scratch = [
        pltpu.VMEM((_B,), jnp.int32),
        pltpu.VMEM((_B,), jnp.int32),
        pltpu.VMEM((_B, 64), jnp.float32),
        pltpu.VMEM((_B, 64), jnp.float32),
        pltpu.VMEM((64, 64), jnp.float32),
        pltpu.VMEM_SHARED((qsz + 8, 64), jnp.float32),
        pltpu.SemaphoreType.DMA,
        pltpu.SemaphoreType.DMA,
    ]

    def body(*refs):
        dstp = refs[0]
        msgs = refs[1:1 + H]
        outs = refs[1 + H:1 + 2 * H]
        (dstv, idxb, rows0, rows1, zbuf, acc, sem0, sem1) = refs[1 + 2 * H:]
        c = lax.axis_index("c")
        s = lax.axis_index("s")
        it = _iota16()
        zf = jnp.zeros((16,), jnp.float32)

        def zz(i, _):
            fl = i * 16 + it
            plsc.store_scatter(zbuf, [fl // 64, fl % 64], zf)
            return 0
        lax.fori_loop(0, 64 * 64 // 16, zz, 0)

        def zacc(k, _):
            pltpu.sync_copy(zbuf, acc.at[pl.ds(s * arpt + k * 64, 64)])
            return 0

        for h in range(H):
            for qq in range(2):
                q = c * 2 + qq
                qbase = q * qsz
                lax.fori_loop(0, arpt // 64, zacc, 0)
                plsc.subcore_barrier()

                def _mkidx(qb):
                    for g in range(_B // 16):
                        d16 = dstv[pl.ds(g * 16, 16)]
                        local = d16 - qb
                        msk = (local >= 0) & (local < qsz)
                        idxb[pl.ds(g * 16, 16)] = jnp.where(msk, local, qsz)

                nh = nbt // 2
                tbase = s * nbt * _B
                pltpu.async_copy(msgs[h].at[pl.ds(tbase, _B)], rows0, sem0)

                def pair(i, _):
                    base0 = tbase + 2 * i * _B
                    base1 = base0 + _B
                    pltpu.async_copy(msgs[h].at[pl.ds(base1, _B)], rows1, sem1)
                    pltpu.sync_copy(dstp.at[pl.ds(base0, _B)], dstv)
                    _mkidx(qbase)
                    pltpu.make_async_copy(msgs[h].at[pl.ds(base0, _B)],
                                          rows0, sem0).wait()
                    pltpu.sync_copy(rows0, acc.at[idxb], add=True)

                    @pl.when(i + 1 < nh)
                    def _():
                        pltpu.async_copy(msgs[h].at[pl.ds(base0 + 2 * _B, _B)],
                                         rows0, sem0)
                    pltpu.sync_copy(dstp.at[pl.ds(base1, _B)], dstv)
                    _mkidx(qbase)
                    pltpu.make_async_copy(msgs[h].at[pl.ds(base1, _B)],
                                          rows1, sem1).wait()
                    pltpu.sync_copy(rows1, acc.at[idxb], add=True)
                    return 0

                lax.fori_loop(0, nh, pair, 0)
                plsc.subcore_barrier()

                def wout(k, _):
                    off = s * arpt + k * 64
                    pltpu.sync_copy(acc.at[pl.ds(off, 64)],
                                    outs[h].at[pl.ds(qbase + off, 64)])
                    return 0
                lax.fori_loop(0, arpt // 64, wout, 0)
                if h + 1 < H or qq == 0:
                    plsc.subcore_barrier()

    return pl.kernel(body, out_type=out_type, mesh=mesh, compiler_params=_CP,
                     scratch_types=scratch, name=f"gat_b_{E_pad}_{H}")


def _ceil_to(x, m):
    return (x + m - 1) // m * m


_BN = 1024


def _tc_grid(n):
    return (n + _BN - 1) // _BN


@functools.cache
def _tc_mm(N, K, M, relu):
    def body(x_ref, w_ref, b_ref, o_ref):
        o = jnp.dot(x_ref[...], w_ref[...],
                    preferred_element_type=jnp.float32) + b_ref[...]
        o_ref[...] = jnp.maximum(o, 0.0) if relu else o

    return pl.pallas_call(
        body,
        grid=(_tc_grid(N),),
        in_specs=[pl.BlockSpec((_BN, K), lambda i: (i, 0)),
                  pl.BlockSpec((K, M), lambda i: (0, 0)),
                  pl.BlockSpec((1, M), lambda i: (0, 0))],
        out_specs=pl.BlockSpec((_BN, M), lambda i: (i, 0)),
        out_shape=jax.ShapeDtypeStruct((N, M), jnp.float32),
        name=f"mm_{N}_{K}_{M}")


def _proj(x, w, b, relu=False):
    return _tc_mm(x.shape[0], x.shape[1], w.shape[1], relu)(x, w, b.reshape(1, -1))


@functools.cache
def _tc_cnn(N):
    def body(x_ref, k_ref, b_ref, s_ref, o_ref):
        h = x_ref[...]
        for i in range(3):
            h = jnp.dot(h, k_ref[i], preferred_element_type=jnp.float32)
            h = jnp.maximum(h + b_ref[i], 0.0)
        pool = jnp.dot(h, s_ref[0], preferred_element_type=jnp.float32)
        for k in range(1, 4):
            pool = jnp.maximum(
                pool, jnp.dot(h, s_ref[k], preferred_element_type=jnp.float32))
        o_ref[...] = pool

    return pl.pallas_call(
        body,
        grid=(_tc_grid(N),),
        in_specs=[pl.BlockSpec((_BN, 125), lambda i: (i, 0)),
                  pl.BlockSpec((3, 125, 125), lambda i: (0, 0, 0)),
                  pl.BlockSpec((3, 1, 125), lambda i: (0, 0, 0)),
                  pl.BlockSpec((4, 125, 45), lambda i: (0, 0, 0))],
        out_specs=pl.BlockSpec((_BN, 45), lambda i: (i, 0)),
        out_shape=jax.ShapeDtypeStruct((N, 45), jnp.float32),
        name="cnn")


@functools.cache
def _pool_sel():
    S = np.zeros((4, 125, 45), np.float32)
    win = [(0,), (1, 2), (3, 4)]
    for c in range(5):
        for i in range(3):
            for j in range(3):
                co = c * 9 + i * 3 + j
                R, C = win[i], win[j]
                for k, (a, b) in enumerate(((0, 0), (0, 1), (1, 0), (1, 1))):
                    r = R[min(a, len(R) - 1)]
                    cc = C[min(b, len(C) - 1)]
                    S[k, c * 25 + r * 5 + cc, co] = 1.0
    return S


def _cnn(x, p):
    mats, biases = [], []
    for i in (1, 2, 3):
        w = p['conv%d_w' % i]
        g, b2, m, v = (p['bn%d_g' % i], p['bn%d_b' % i],
                       p['bn%d_m' % i], p['bn%d_v' % i])
        scale = g / jnp.sqrt(v + 1e-5)
        shift = b2 - m * scale
        PI, PO, CC, DI, DJ = [], [], [], [], []
        for c in range(5):
            for ri in range(5):
                for cj in range(5):
                    for di in range(3):
                        for dj in range(3):
                            ii, jj = ri - di + 1, cj - dj + 1
                            if 0 <= ii < 5 and 0 <= jj < 5:
                                PI.append(c * 25 + ri * 5 + cj)
                                PO.append(c * 25 + ii * 5 + jj)
                                CC.append(c)
                                DI.append(di)
                                DJ.append(dj)
        K = jnp.zeros((125, 125), jnp.float32).at[
            np.array(PI), np.array(PO)].set(w[np.array(CC), 0, np.array(DI), np.array(DJ)])
        colscale = jnp.repeat(scale, 25)
        mats.append(K * colscale[None, :])
        biases.append((jnp.repeat(p['conv%d_b' % i], 25) * colscale
                       + jnp.repeat(shift, 25)).reshape(1, 125))
    kmat = jnp.stack(mats)
    bmat = jnp.stack(biases)
    S = jnp.asarray(_pool_sel())
    x2 = x.reshape(x.shape[0], 125)
    return _tc_cnn(x2.shape[0])(x2, kmat, bmat, S)


@functools.cache
def _tc_finish(N, H, relu, bn):
    F = H * 64

    def body(*refs):
        s_refs = refs[:H]
        d0_ref, d1_ref, bias_ref, sc_ref, sh_ref, o_ref = refs[H:]
        cnt = jnp.maximum(d0_ref[:, 2:3] + d1_ref[:, 2:3], 1.0)
        cols = []
        for h in range(H):
            den = d0_ref[:, h:h + 1] + d1_ref[:, h:h + 1]
            cols.append(s_refs[h][...] / (den * cnt + 1e-30))
        o = jnp.concatenate(cols, axis=1) + bias_ref[...]
        if bn:
            o = o * sc_ref[...] + sh_ref[...]
        o_ref[...] = jnp.maximum(o, 0.0) if relu else o

    return pl.pallas_call(
        body,
        grid=(_tc_grid(N),),
        in_specs=[pl.BlockSpec((_BN, 64), lambda i: (i, 0)) for _ in range(H)]
        + [pl.BlockSpec((_BN, 8), lambda i: (i, 0)),
           pl.BlockSpec((_BN, 8), lambda i: (i, 0)),
           pl.BlockSpec((1, F), lambda i: (0, 0)),
           pl.BlockSpec((1, F), lambda i: (0, 0)),
           pl.BlockSpec((1, F), lambda i: (0, 0))],
        out_specs=pl.BlockSpec((_BN, F), lambda i: (i, 0)),
        out_shape=jax.ShapeDtypeStruct((N, F), jnp.float32),
        name=f"finish_{N}_{H}")


@functools.cache
def _tc_mlp(N):
    def body(x_ref, w1, b1, w2, b2, w3, b3, o_ref):
        h = jnp.maximum(jnp.dot(x_ref[...], w1[...],
                                preferred_element_type=jnp.float32) + b1[...], 0.0)
        h = jnp.maximum(jnp.dot(h, w2[...],
                                preferred_element_type=jnp.float32) + b2[...], 0.0)
        o_ref[...] = jnp.dot(h, w3[...],
                             preferred_element_type=jnp.float32) + b3[...]

    return pl.pallas_call(
        body,
        grid=(_tc_grid(N),),
        in_specs=[pl.BlockSpec((_BN, 64), lambda i: (i, 0)),
                  pl.BlockSpec((64, 64), lambda i: (0, 0)),
                  pl.BlockSpec((1, 64), lambda i: (0, 0)),
                  pl.BlockSpec((64, 32), lambda i: (0, 0)),
                  pl.BlockSpec((1, 32), lambda i: (0, 0)),
                  pl.BlockSpec((32, 1), lambda i: (0, 0)),
                  pl.BlockSpec((1, 1), lambda i: (0, 0))],
        out_specs=pl.BlockSpec((_BN, 1), lambda i: (i, 0)),
        out_shape=jax.ShapeDtypeStruct((N, 1), jnp.float32),
        name="mlp")


def _pad1(a, n):
    return jnp.concatenate([a, jnp.zeros((n - a.shape[0],), a.dtype)])


def _sc_gat(x_src, x_dst, ei, p, name, heads, num_dst, bn=None, relu=False):
    if x_src is x_dst:
        w2 = jnp.concatenate([p[name + '_Wl'], p[name + '_Wr']], axis=1)
        b2 = jnp.concatenate([p[name + '_bl'], p[name + '_br']])
        xlr = _proj(x_src, w2, b2)
        F = heads * 64
        xl, xr = xlr[:, :F], xlr[:, F:]
    else:
        xl = _proj(x_src, p[name + '_Wl'], p[name + '_bl'])
        xr = _proj(x_dst, p[name + '_Wr'], p[name + '_br'])
    F = heads * 64
    E = ei.shape[1]
    E_pad = _ceil_to(E, _NW * _B)
    N_pad = _ceil_to(num_dst, 4 * _NS * 64)
    srcp = _pad1(ei[0], E_pad)
    dstp = _pad1(ei[1], E_pad)
    att = p[name + '_att'].reshape(-1)
    res = _phase_a(E, E_pad, F, heads, N_pad)(xl, xr, att, srcp, dstp)
    den2, msgs = res[0], res[1:]
    den2 = den2.reshape(_NC, N_pad, 8)
    if bn is not None:
        g, b2_, m, v = bn
        scale = g / jnp.sqrt(v + 1e-5)
        shift = b2_ - m * scale
    else:
        scale = jnp.ones((F,), jnp.float32)
        shift = jnp.zeros((F,), jnp.float32)
    outs = _phase_b(E_pad, heads, N_pad)(dstp, *msgs)
    out = _tc_finish(N_pad, heads, relu, bn is not None)(
        *outs, den2[0], den2[1], p[name + '_bias'].reshape(1, -1),
        scale.reshape(1, -1), shift.reshape(1, -1))
    return out[:num_dst]


def kernel(x_low, x_9x, x_25x, x_high, z_std_high, ei_low_9x, ei_9x_25x, ei_25x_high, ei_high, params):
    p = params
    h = _cnn(x_low, p)
    h = _sc_gat(h, x_9x, ei_low_9x, p, 'd1', 1, x_9x.shape[0])
    h = _sc_gat(h, x_25x, ei_9x_25x, p, 'd2', 1, x_25x.shape[0])
    h = _sc_gat(h, x_high, ei_25x_high, p, 'd3', 1, x_high.shape[0])
    h = jnp.concatenate([z_std_high, h], axis=-1)
    n = h.shape[0]
    loops = jnp.arange(n, dtype=ei_high.dtype)
    ei = jnp.concatenate([ei_high, jnp.stack([loops, loops])], axis=1)
    for i in (1, 2, 3, 4):
        bn = (p['pbn%d_g' % i], p['pbn%d_b' % i], p['pbn%d_m' % i], p['pbn%d_v' % i])
        h = _sc_gat(h, h, ei, p, 'p%d' % i, 2, n, bn=bn, relu=True)
    h = _sc_gat(h, h, ei, p, 'p5', 1, n, relu=True)
    return _tc_mlp(n)(h, p['fc1_w'], p['fc1_b'].reshape(1, -1),
                      p['fc2_w'], p['fc2_b'].reshape(1, -1),
                      p['fc3_w'], p['fc3_b'].reshape(1, -1))


# speedup vs baseline: 1.9977x; 1.9890x over previous
import functools

import numpy as np

import jax
import jax.numpy as jnp
from jax import lax
from jax.experimental import pallas as pl
from jax.experimental.pallas import tpu as pltpu
from jax.experimental.pallas import tpu_sc as plsc

_B = 128
_NC = 2
_NS = 16
_NW = _NC * _NS
_CP = pltpu.CompilerParams(use_tc_tiling_on_sc=False, needs_layout_passes=False)


def _iota16():
    return lax.iota(jnp.int32, 16)


def _splat_i(x):
    return jnp.full((16,), x, jnp.int32)


@functools.cache
def _phase_a(E_real, E_pad, F, H, N_pad):
    mesh = plsc.VectorSubcoreMesh(core_axis_name="c", subcore_axis_name="s",
                                  num_cores=_NC, num_subcores=_NS)
    nblk = E_pad // (_NW * _B)
    drpt = N_pad // _NS
    out_type = tuple([jax.ShapeDtypeStruct((_NC * N_pad, 8), jnp.float32)] +
                     [jax.ShapeDtypeStruct((E_pad, 64), jnp.float32)
                      for _ in range(H)])
    scratch = ([
        pltpu.VMEM((_B,), jnp.int32),
        pltpu.VMEM((_B,), jnp.int32),
        pltpu.VMEM((_B, F), jnp.float32),
        pltpu.VMEM((_B, F), jnp.float32),
        pltpu.VMEM((_B, 8), jnp.float32),
        pltpu.VMEM((F,), jnp.float32),
        pltpu.VMEM((_B, 8), jnp.float32),
        pltpu.VMEM_SHARED((N_pad, 8), jnp.float32),
    ] + [pltpu.VMEM((_B, 64), jnp.float32) for _ in range(H)]
      + [pltpu.VMEM((_B,), jnp.float32) for _ in range(H)]
      + [pltpu.SemaphoreType.DMA, pltpu.SemaphoreType.DMA])

    def body(*refs):
        (xl, xr, att, srcp, dstp, den2_o) = refs[:6]
        msg_o = refs[6:6 + H]
        (srcv, dstv, rows_l, rows_r, denblk, attv, zbuf, dacc) = refs[6 + H:14 + H]
        msgb = refs[14 + H:14 + 2 * H]
        exb = refs[14 + 2 * H:14 + 3 * H]
        sem1, sem2 = refs[14 + 3 * H:]
        c = lax.axis_index("c")
        s = lax.axis_index("s")
        wid = s * _NC + c
        it = _iota16()
        zf = jnp.zeros((16,), jnp.float32)
        for col in range(3):
            for r in range(_B // 16):
                plsc.store_scatter(zbuf, [r * 16 + it, _splat_i(col)], zf)
                plsc.store_scatter(denblk, [r * 16 + it, _splat_i(col)], zf)
        def zden(k, _):
            pltpu.sync_copy(zbuf, dacc.at[pl.ds(s * drpt + k * _B, _B)])
            return 0
        lax.fori_loop(0, drpt // _B, zden, 0)
        pltpu.sync_copy(att, attv)
        plsc.subcore_barrier()

        def block(i, _):
            base = (wid * nblk + i) * _B
            pltpu.sync_copy(srcp.at[pl.ds(base, _B)], srcv)
            pltpu.sync_copy(dstp.at[pl.ds(base, _B)], dstv)
            cp1 = pltpu.async_copy(xl.at[srcv], rows_l, sem1)
            cp2 = pltpu.async_copy(xr.at[dstv], rows_r, sem2)
            att_k = [attv[pl.ds(kk * 16, 16)] for kk in range(F // 16)]
            cp1.wait()
            cp2.wait()

            def jbody(j, _):
                for h in range(H):
                    acc = zf
                    vls = []
                    for kk in range(4):
                        col = h * 64 + kk * 16
                        vl = rows_l[j, pl.ds(col, 16)]
                        vr = rows_r[j, pl.ds(col, 16)]
                        vls.append(vl)
                        sm = vl + vr
                        e = jnp.maximum(sm, 0.2 * sm)
                        acc = acc + e * att_k[h * 4 + kk]
                    sc = jnp.sum(acc)
                    m16 = jnp.full((16,), base + j < E_real)
                    ex = jnp.where(
                        m16, jnp.exp(jnp.full((16,), sc, jnp.float32)), zf)
                    plsc.store_scatter(exb[h], [_splat_i(j)], ex)
                    for kk in range(4):
                        msgb[h][j, pl.ds(kk * 16, 16)] = vls[kk] * ex
                return 0
            lax.fori_loop(0, _B, jbody, 0)
            for g in range(_B // 16):
                rowi = g * 16 + it
                eids = base + rowi
                cntv = jnp.where(eids < E_real, 1.0, 0.0)
                plsc.store_scatter(denblk, [rowi, _splat_i(2)], cntv)
                for h in range(H):
                    ex16 = exb[h][pl.ds(g * 16, 16)]
                    plsc.store_scatter(denblk, [rowi, _splat_i(h)], ex16)
            for h in range(H):
                pltpu.sync_copy(msgb[h], msg_o[h].at[pl.ds(base, _B)])
            pltpu.sync_copy(denblk, dacc.at[dstv], add=True)
            return 0

        lax.fori_loop(0, nblk, block, 0)
        plsc.subcore_barrier()

        def wout(k, _):
            off = s * drpt + k * _B
            pltpu.sync_copy(dacc.at[pl.ds(off, _B)],
                            den2_o.at[pl.ds(c * N_pad + off, _B)])
            return 0
        lax.fori_loop(0, drpt // _B, wout, 0)

    return pl.kernel(body, out_type=out_type, mesh=mesh, compiler_params=_CP,
                     scratch_types=scratch, name=f"gat_a_{E_pad}_{F}_{H}")


@functools.cache
def _phase_b(E_pad, H, N_pad):
    mesh = plsc.VectorSubcoreMesh(core_axis_name="c", subcore_axis_name="s",
                                  num_cores=_NC, num_subcores=_NS)
    qsz = N_pad // 4
    arpt = qsz // _NS
    nbt = E_pad // (_NS * _B)
    out_type = tuple(jax.ShapeDtypeStruct((N_pad, 64), jnp.float32)
                     for _ in range(H))
    scratch = [
        pltpu.VMEM((_B,), jnp.int32),
        pltpu.VMEM((_B,), jnp.int32),
        pltpu.VMEM((_B, 64), jnp.float32),
        pltpu.VMEM((_B, 64), jnp.float32),
        pltpu.VMEM((64, 64), jnp.float32),
        pltpu.VMEM_SHARED((qsz + 8, 64), jnp.float32),
        pltpu.SemaphoreType.DMA,
        pltpu.SemaphoreType.DMA,
    ]

    def body(*refs):
        dstp = refs[0]
        msgs = refs[1:1 + H]
        outs = refs[1 + H:1 + 2 * H]
        (dstv, idxb, rows0, rows1, zbuf, acc, sem0, sem1) = refs[1 + 2 * H:]
        c = lax.axis_index("c")
        s = lax.axis_index("s")
        it = _iota16()
        zf = jnp.zeros((16,), jnp.float32)

        def zz(i, _):
            fl = i * 16 + it
            plsc.store_scatter(zbuf, [fl // 64, fl % 64], zf)
            return 0
        lax.fori_loop(0, 64 * 64 // 16, zz, 0)

        def zacc(k, _):
            pltpu.sync_copy(zbuf, acc.at[pl.ds(s * arpt + k * 64, 64)])
            return 0

        for h in range(H):
            for qq in range(2):
                q = c * 2 + qq
                qbase = q * qsz
                lax.fori_loop(0, arpt // 64, zacc, 0)
                plsc.subcore_barrier()

                def _mkidx(qb):
                    for g in range(_B // 16):
                        d16 = dstv[pl.ds(g * 16, 16)]
                        local = d16 - qb
                        msk = (local >= 0) & (local < qsz)
                        idxb[pl.ds(g * 16, 16)] = jnp.where(msk, local, qsz)

                nh = nbt // 2
                tbase = s * nbt * _B
                pltpu.async_copy(msgs[h].at[pl.ds(tbase, _B)], rows0, sem0)

                def pair(i, _):
                    base0 = tbase + 2 * i * _B
                    base1 = base0 + _B
                    pltpu.async_copy(msgs[h].at[pl.ds(base1, _B)], rows1, sem1)
                    pltpu.sync_copy(dstp.at[pl.ds(base0, _B)], dstv)
                    _mkidx(qbase)
                    pltpu.make_async_copy(msgs[h].at[pl.ds(base0, _B)],
                                          rows0, sem0).wait()
                    pltpu.sync_copy(rows0, acc.at[idxb], add=True)

                    @pl.when(i + 1 < nh)
                    def _():
                        pltpu.async_copy(msgs[h].at[pl.ds(base0 + 2 * _B, _B)],
                                         rows0, sem0)
                    pltpu.sync_copy(dstp.at[pl.ds(base1, _B)], dstv)
                    _mkidx(qbase)
                    pltpu.make_async_copy(msgs[h].at[pl.ds(base1, _B)],
                                          rows1, sem1).wait()
                    pltpu.sync_copy(rows1, acc.at[idxb], add=True)
                    return 0

                lax.fori_loop(0, nh, pair, 0)
                plsc.subcore_barrier()

                def wout(k, _):
                    off = s * arpt + k * 64
                    pltpu.sync_copy(acc.at[pl.ds(off, 64)],
                                    outs[h].at[pl.ds(qbase + off, 64)])
                    return 0
                lax.fori_loop(0, arpt // 64, wout, 0)
                if h + 1 < H or qq == 0:
                    plsc.subcore_barrier()

    return pl.kernel(body, out_type=out_type, mesh=mesh, compiler_params=_CP,
                     scratch_types=scratch, name=f"gat_b_{E_pad}_{H}")


def _ceil_to(x, m):
    return (x + m - 1) // m * m


_BN = 1024


def _tc_grid(n):
    return (n + _BN - 1) // _BN


@functools.cache
def _tc_mm(N, K, M, relu):
    def body(x_ref, w_ref, b_ref, o_ref):
        o = jnp.dot(x_ref[...], w_ref[...],
                    preferred_element_type=jnp.float32) + b_ref[...]
        o_ref[...] = jnp.maximum(o, 0.0) if relu else o

    return pl.pallas_call(
        body,
        grid=(_tc_grid(N),),
        in_specs=[pl.BlockSpec((_BN, K), lambda i: (i, 0)),
                  pl.BlockSpec((K, M), lambda i: (0, 0)),
                  pl.BlockSpec((1, M), lambda i: (0, 0))],
        out_specs=pl.BlockSpec((_BN, M), lambda i: (i, 0)),
        out_shape=jax.ShapeDtypeStruct((N, M), jnp.float32),
        name=f"mm_{N}_{K}_{M}")


def _proj(x, w, b, relu=False):
    return _tc_mm(x.shape[0], x.shape[1], w.shape[1], relu)(x, w, b.reshape(1, -1))


@functools.cache
def _tc_cnn(N):
    def body(x_ref, k_ref, b_ref, s_ref, o_ref):
        h = x_ref[...]
        for i in range(3):
            h = jnp.dot(h, k_ref[i], preferred_element_type=jnp.float32)
            h = jnp.maximum(h + b_ref[i], 0.0)
        pool = jnp.dot(h, s_ref[0], preferred_element_type=jnp.float32)
        for k in range(1, 4):
            pool = jnp.maximum(
                pool, jnp.dot(h, s_ref[k], preferred_element_type=jnp.float32))
        o_ref[...] = pool

    return pl.pallas_call(
        body,
        grid=(_tc_grid(N),),
        in_specs=[pl.BlockSpec((_BN, 125), lambda i: (i, 0)),
                  pl.BlockSpec((3, 125, 125), lambda i: (0, 0, 0)),
                  pl.BlockSpec((3, 1, 125), lambda i: (0, 0, 0)),
                  pl.BlockSpec((4, 125, 45), lambda i: (0, 0, 0))],
        out_specs=pl.BlockSpec((_BN, 45), lambda i: (i, 0)),
        out_shape=jax.ShapeDtypeStruct((N, 45), jnp.float32),
        name="cnn")


@functools.cache
def _pool_sel():
    S = np.zeros((4, 125, 45), np.float32)
    win = [(0,), (1, 2), (3, 4)]
    for c in range(5):
        for i in range(3):
            for j in range(3):
                co = c * 9 + i * 3 + j
                R, C = win[i], win[j]
                for k, (a, b) in enumerate(((0, 0), (0, 1), (1, 0), (1, 1))):
                    r = R[min(a, len(R) - 1)]
                    cc = C[min(b, len(C) - 1)]
                    S[k, c * 25 + r * 5 + cc, co] = 1.0
    return S


def _cnn(x, p):
    mats, biases = [], []
    for i in (1, 2, 3):
        w = p['conv%d_w' % i]
        g, b2, m, v = (p['bn%d_g' % i], p['bn%d_b' % i],
                       p['bn%d_m' % i], p['bn%d_v' % i])
        scale = g / jnp.sqrt(v + 1e-5)
        shift = b2 - m * scale
        PI, PO, CC, DI, DJ = [], [], [], [], []
        for c in range(5):
            for ri in range(5):
                for cj in range(5):
                    for di in range(3):
                        for dj in range(3):
                            ii, jj = ri - di + 1, cj - dj + 1
                            if 0 <= ii < 5 and 0 <= jj < 5:
                                PI.append(c * 25 + ri * 5 + cj)
                                PO.append(c * 25 + ii * 5 + jj)
                                CC.append(c)
                                DI.append(di)
                                DJ.append(dj)
        K = jnp.zeros((125, 125), jnp.float32).at[
            np.array(PI), np.array(PO)].set(w[np.array(CC), 0, np.array(DI), np.array(DJ)])
        colscale = jnp.repeat(scale, 25)
        mats.append(K * colscale[None, :])
        biases.append((jnp.repeat(p['conv%d_b' % i], 25) * colscale
                       + jnp.repeat(shift, 25)).reshape(1, 125))
    kmat = jnp.stack(mats)
    bmat = jnp.stack(biases)
    S = jnp.asarray(_pool_sel())
    x2 = x.reshape(x.shape[0], 125)
    return _tc_cnn(x2.shape[0])(x2, kmat, bmat, S)


@functools.cache
def _tc_finish(N, H, relu, bn):
    F = H * 64

    def body(*refs):
        s_refs = refs[:H]
        d0_ref, d1_ref, bias_ref, sc_ref, sh_ref, o_ref = refs[H:]
        cnt = jnp.maximum(d0_ref[:, 2:3] + d1_ref[:, 2:3], 1.0)
        cols = []
        for h in range(H):
            den = d0_ref[:, h:h + 1] + d1_ref[:, h:h + 1]
            cols.append(s_refs[h][...] / (den * cnt + 1e-30))
        o = jnp.concatenate(cols, axis=1) + bias_ref[...]
        if bn:
            o = o * sc_ref[...] + sh_ref[...]
        o_ref[...] = jnp.maximum(o, 0.0) if relu else o

    return pl.pallas_call(
        body,
        grid=(_tc_grid(N),),
        in_specs=[pl.BlockSpec((_BN, 64), lambda i: (i, 0)) for _ in range(H)]
        + [pl.BlockSpec((_BN, 8), lambda i: (i, 0)),
           pl.BlockSpec((_BN, 8), lambda i: (i, 0)),
           pl.BlockSpec((1, F), lambda i: (0, 0)),
           pl.BlockSpec((1, F), lambda i: (0, 0)),
           pl.BlockSpec((1, F), lambda i: (0, 0))],
        out_specs=pl.BlockSpec((_BN, F), lambda i: (i, 0)),
        out_shape=jax.ShapeDtypeStruct((N, F), jnp.float32),
        name=f"finish_{N}_{H}")


@functools.cache
def _tc_mlp(N):
    def body(x_ref, w1, b1, w2, b2, w3, b3, o_ref):
        h = jnp.maximum(jnp.dot(x_ref[...], w1[...],
                                preferred_element_type=jnp.float32) + b1[...], 0.0)
        h = jnp.maximum(jnp.dot(h, w2[...],
                                preferred_element_type=jnp.float32) + b2[...], 0.0)
        o_ref[...] = jnp.dot(h, w3[...],
                             preferred_element_type=jnp.float32) + b3[...]

    return pl.pallas_call(
        body,
        grid=(_tc_grid(N),),
        in_specs=[pl.BlockSpec((_BN, 64), lambda i: (i, 0)),
                  pl.BlockSpec((64, 64), lambda i: (0, 0)),
                  pl.BlockSpec((1, 64), lambda i: (0, 0)),
                  pl.BlockSpec((64, 32), lambda i: (0, 0)),
                  pl.BlockSpec((1, 32), lambda i: (0, 0)),
                  pl.BlockSpec((32, 1), lambda i: (0, 0)),
                  pl.BlockSpec((1, 1), lambda i: (0, 0))],
        out_specs=pl.BlockSpec((_BN, 1), lambda i: (i, 0)),
        out_shape=jax.ShapeDtypeStruct((N, 1), jnp.float32),
        name="mlp")


def _pad1(a, n):
    return jnp.concatenate([a, jnp.zeros((n - a.shape[0],), a.dtype)])


def _sc_gat(x_src, x_dst, ei, p, name, heads, num_dst, bn=None, relu=False):
    if x_src is x_dst:
        w2 = jnp.concatenate([p[name + '_Wl'], p[name + '_Wr']], axis=1)
        b2 = jnp.concatenate([p[name + '_bl'], p[name + '_br']])
        xlr = _proj(x_src, w2, b2)
        F = heads * 64
        xl, xr = xlr[:, :F], xlr[:, F:]
    else:
        xl = _proj(x_src, p[name + '_Wl'], p[name + '_bl'])
        xr = _proj(x_dst, p[name + '_Wr'], p[name + '_br'])
    F = heads * 64
    E = ei.shape[1]
    E_pad = _ceil_to(E, _NW * _B)
    N_pad = _ceil_to(num_dst, 4 * _NS * 64)
    srcp = _pad1(ei[0], E_pad)
    dstp = _pad1(ei[1], E_pad)
    att = p[name + '_att'].reshape(-1)
    res = _phase_a(E, E_pad, F, heads, N_pad)(xl, xr, att, srcp, dstp)
    den2, msgs = res[0], res[1:]
    den2 = den2.reshape(_NC, N_pad, 8)
    if bn is not None:
        g, b2_, m, v = bn
        scale = g / jnp.sqrt(v + 1e-5)
        shift = b2_ - m * scale
    else:
        scale = jnp.ones((F,), jnp.float32)
        shift = jnp.zeros((F,), jnp.float32)
    outs = _phase_b(E_pad, heads, N_pad)(dstp, *msgs)
    out = _tc_finish(N_pad, heads, relu, bn is not None)(
        *outs, den2[0], den2[1], p[name + '_bias'].reshape(1, -1),
        scale.reshape(1, -1), shift.reshape(1, -1))
    return out[:num_dst]


def kernel(x_low, x_9x, x_25x, x_high, z_std_high, ei_low_9x, ei_9x_25x, ei_25x_high, ei_high, params):
    p = params
    h = _cnn(x_low, p)
    h = _sc_gat(h, x_9x, ei_low_9x, p, 'd1', 1, x_9x.shape[0])
    h = _sc_gat(h, x_25x, ei_9x_25x, p, 'd2', 1, x_25x.shape[0])
    h = _sc_gat(h, x_high, ei_25x_high, p, 'd3', 1, x_high.shape[0])
    h = jnp.concatenate([z_std_high, h], axis=-1)
    n = h.shape[0]
    loops = jnp.arange(n, dtype=ei_high.dtype)
    ei = jnp.concatenate([ei_high, jnp.stack([loops, loops])], axis=1)
    for i in (1, 2, 3, 4):
        bn = (p['pbn%d_g' % i], p['pbn%d_b' % i], p['pbn%d_m' % i], p['pbn%d_v' % i])
        h = _sc_gat(h, h, ei, p, 'p%d' % i, 2, n, bn=bn, relu=True)
    h = _sc_gat(h, h, ei, p, 'p5', 1, n, relu=True)
    return _tc_mlp(n)(h, p['fc1_w'], p['fc1_b'].reshape(1, -1),
                      p['fc2_w'], p['fc2_b'].reshape(1, -1),
                      p['fc3_w'], p['fc3_b'].reshape(1, -1))


# speedup vs baseline: 2.0100x; 1.0062x over previous
import functools

import numpy as np

import jax
import jax.numpy as jnp
from jax import lax
from jax.experimental import pallas as pl
from jax.experimental.pallas import tpu as pltpu
from jax.experimental.pallas import tpu_sc as plsc

_B = 128
_NC = 2
_NS = 16
_NW = _NC * _NS
_CP = pltpu.CompilerParams(use_tc_tiling_on_sc=False, needs_layout_passes=False)


def _iota16():
    return lax.iota(jnp.int32, 16)


def _splat_i(x):
    return jnp.full((16,), x, jnp.int32)


@functools.cache
def _phase_a(E_real, E_pad, F, H, N_pad):
    mesh = plsc.VectorSubcoreMesh(core_axis_name="c", subcore_axis_name="s",
                                  num_cores=_NC, num_subcores=_NS)
    nblk = E_pad // (_NW * _B)
    drpt = N_pad // _NS
    out_type = tuple([jax.ShapeDtypeStruct((_NC * N_pad, 8), jnp.float32)] +
                     [jax.ShapeDtypeStruct((E_pad, 64), jnp.float32)
                      for _ in range(H)])
    scratch = ([
        pltpu.VMEM((_B,), jnp.int32),
        pltpu.VMEM((_B,), jnp.int32),
        pltpu.VMEM((_B, F), jnp.float32),
        pltpu.VMEM((_B, F), jnp.float32),
        pltpu.VMEM((_B, 8), jnp.float32),
        pltpu.VMEM((F,), jnp.float32),
        pltpu.VMEM((_B, 8), jnp.float32),
        pltpu.VMEM_SHARED((N_pad, 8), jnp.float32),
    ] + [pltpu.VMEM((_B, 64), jnp.float32) for _ in range(H)]
      + [pltpu.VMEM((_B,), jnp.float32) for _ in range(H)]
      + [pltpu.SemaphoreType.DMA, pltpu.SemaphoreType.DMA])

    def body(*refs):
        (xl, xr, att, srcp, dstp, den2_o) = refs[:6]
        msg_o = refs[6:6 + H]
        (srcv, dstv, rows_l, rows_r, denblk, attv, zbuf, dacc) = refs[6 + H:14 + H]
        msgb = refs[14 + H:14 + 2 * H]
        exb = refs[14 + 2 * H:14 + 3 * H]
        sem1, sem2 = refs[14 + 3 * H:]
        c = lax.axis_index("c")
        s = lax.axis_index("s")
        wid = s * _NC + c
        it = _iota16()
        zf = jnp.zeros((16,), jnp.float32)
        for col in range(3):
            for r in range(_B // 16):
                plsc.store_scatter(zbuf, [r * 16 + it, _splat_i(col)], zf)
                plsc.store_scatter(denblk, [r * 16 + it, _splat_i(col)], zf)
        def zden(k, _):
            pltpu.sync_copy(zbuf, dacc.at[pl.ds(s * drpt + k * _B, _B)])
            return 0
        lax.fori_loop(0, drpt // _B, zden, 0)
        pltpu.sync_copy(att, attv)
        plsc.subcore_barrier()

        def block(i, _):
            base = (wid * nblk + i) * _B
            pltpu.sync_copy(srcp.at[pl.ds(base, _B)], srcv)
            pltpu.sync_copy(dstp.at[pl.ds(base, _B)], dstv)
            cp1 = pltpu.async_copy(xl.at[srcv], rows_l, sem1)
            cp2 = pltpu.async_copy(xr.at[dstv], rows_r, sem2)
            att_k = [attv[pl.ds(kk * 16, 16)] for kk in range(F // 16)]
            cp1.wait()
            cp2.wait()

            def jbody(jj, _):
                for u in range(2):
                    j = jj * 2 + u
                    for h in range(H):
                        acc = zf
                        vls = []
                        for kk in range(4):
                            col = h * 64 + kk * 16
                            vl = rows_l[j, pl.ds(col, 16)]
                            vr = rows_r[j, pl.ds(col, 16)]
                            vls.append(vl)
                            sm = vl + vr
                            e = jnp.maximum(sm, 0.2 * sm)
                            acc = acc + e * att_k[h * 4 + kk]
                        sc = jnp.sum(acc)
                        m16 = jnp.full((16,), base + j < E_real)
                        ex = jnp.where(
                            m16, jnp.exp(jnp.full((16,), sc, jnp.float32)), zf)
                        plsc.store_scatter(exb[h], [_splat_i(j)], ex)
                        for kk in range(4):
                            msgb[h][j, pl.ds(kk * 16, 16)] = vls[kk] * ex
                return 0
            lax.fori_loop(0, _B // 2, jbody, 0)
            for g in range(_B // 16):
                rowi = g * 16 + it
                eids = base + rowi
                cntv = jnp.where(eids < E_real, 1.0, 0.0)
                plsc.store_scatter(denblk, [rowi, _splat_i(2)], cntv)
                for h in range(H):
                    ex16 = exb[h][pl.ds(g * 16, 16)]
                    plsc.store_scatter(denblk, [rowi, _splat_i(h)], ex16)
            for h in range(H):
                pltpu.sync_copy(msgb[h], msg_o[h].at[pl.ds(base, _B)])
            pltpu.sync_copy(denblk, dacc.at[dstv], add=True)
            return 0

        lax.fori_loop(0, nblk, block, 0)
        plsc.subcore_barrier()

        def wout(k, _):
            off = s * drpt + k * _B
            pltpu.sync_copy(dacc.at[pl.ds(off, _B)],
                            den2_o.at[pl.ds(c * N_pad + off, _B)])
            return 0
        lax.fori_loop(0, drpt // _B, wout, 0)

    return pl.kernel(body, out_type=out_type, mesh=mesh, compiler_params=_CP,
                     scratch_types=scratch, name=f"gat_a_{E_pad}_{F}_{H}")


@functools.cache
def _phase_b(E_pad, H, N_pad):
    mesh = plsc.VectorSubcoreMesh(core_axis_name="c", subcore_axis_name="s",
                                  num_cores=_NC, num_subcores=_NS)
    qsz = N_pad // 4
    arpt = qsz // _NS
    nbt = E_pad // (_NS * _B)
    out_type = tuple(jax.ShapeDtypeStruct((N_pad, 64), jnp.float32)
                     for _ in range(H))
    scratch = [
        pltpu.VMEM((_B,), jnp.int32),
        pltpu.VMEM((_B,), jnp.int32),
        pltpu.VMEM((_B,), jnp.int32),
        pltpu.VMEM((_B, 64), jnp.float32),
        pltpu.VMEM((_B, 64), jnp.float32),
        pltpu.VMEM((64, 64), jnp.float32),
        pltpu.VMEM_SHARED((qsz + 8, 64), jnp.float32),
        pltpu.SemaphoreType.DMA,
        pltpu.SemaphoreType.DMA,
        pltpu.SemaphoreType.DMA,
        pltpu.SemaphoreType.DMA,
    ]

    def body(*refs):
        dstp = refs[0]
        msgs = refs[1:1 + H]
        outs = refs[1 + H:1 + 2 * H]
        (dstv, idxb0, idxb1, rows0, rows1, zbuf, acc,
         sem0, sem1, semS0, semS1) = refs[1 + 2 * H:]
        c = lax.axis_index("c")
        s = lax.axis_index("s")
        it = _iota16()
        zf = jnp.zeros((16,), jnp.float32)

        def zz(i, _):
            fl = i * 16 + it
            plsc.store_scatter(zbuf, [fl // 64, fl % 64], zf)
            return 0
        lax.fori_loop(0, 64 * 64 // 16, zz, 0)

        def zacc(k, _):
            pltpu.sync_copy(zbuf, acc.at[pl.ds(s * arpt + k * 64, 64)])
            return 0

        for h in range(H):
            for qq in range(2):
                q = c * 2 + qq
                qbase = q * qsz
                lax.fori_loop(0, arpt // 64, zacc, 0)
                plsc.subcore_barrier()

                def _mkidx(qb, ib):
                    for g in range(_B // 16):
                        d16 = dstv[pl.ds(g * 16, 16)]
                        local = d16 - qb
                        msk = (local >= 0) & (local < qsz)
                        ib[pl.ds(g * 16, 16)] = jnp.where(msk, local, qsz)

                nh = nbt // 2
                tbase = s * nbt * _B
                pltpu.async_copy(msgs[h].at[pl.ds(tbase, _B)], rows0, sem0)

                def pair(i, _):
                    base0 = tbase + 2 * i * _B
                    base1 = base0 + _B

                    @pl.when(i > 0)
                    def _():
                        pltpu.make_async_copy(rows1, acc.at[idxb1],
                                              semS1).wait()
                    pltpu.async_copy(msgs[h].at[pl.ds(base1, _B)], rows1, sem1)

                    pltpu.sync_copy(dstp.at[pl.ds(base0, _B)], dstv)
                    _mkidx(qbase, idxb0)
                    pltpu.make_async_copy(msgs[h].at[pl.ds(base0, _B)],
                                          rows0, sem0).wait()
                    pltpu.async_copy(rows0, acc.at[idxb0], semS0, add=True)

                    pltpu.sync_copy(dstp.at[pl.ds(base1, _B)], dstv)
                    _mkidx(qbase, idxb1)
                    pltpu.make_async_copy(msgs[h].at[pl.ds(base1, _B)],
                                          rows1, sem1).wait()
                    pltpu.async_copy(rows1, acc.at[idxb1], semS1, add=True)

                    @pl.when(i + 1 < nh)
                    def _():
                        pltpu.make_async_copy(rows0, acc.at[idxb0],
                                              semS0).wait()
                        pltpu.async_copy(msgs[h].at[pl.ds(base0 + 2 * _B, _B)],
                                         rows0, sem0)
                    return 0

                lax.fori_loop(0, nh, pair, 0)
                pltpu.make_async_copy(rows0, acc.at[idxb0], semS0).wait()
                pltpu.make_async_copy(rows1, acc.at[idxb1], semS1).wait()
                plsc.subcore_barrier()

                def wout(k, _):
                    off = s * arpt + k * 64
                    pltpu.sync_copy(acc.at[pl.ds(off, 64)],
                                    outs[h].at[pl.ds(qbase + off, 64)])
                    return 0
                lax.fori_loop(0, arpt // 64, wout, 0)
                if h + 1 < H or qq == 0:
                    plsc.subcore_barrier()

    return pl.kernel(body, out_type=out_type, mesh=mesh, compiler_params=_CP,
                     scratch_types=scratch, name=f"gat_b_{E_pad}_{H}")


def _ceil_to(x, m):
    return (x + m - 1) // m * m


_BN = 1024


def _tc_grid(n):
    return (n + _BN - 1) // _BN


@functools.cache
def _tc_mm(N, K, M, relu):
    def body(x_ref, w_ref, b_ref, o_ref):
        o = jnp.dot(x_ref[...], w_ref[...],
                    preferred_element_type=jnp.float32) + b_ref[...]
        o_ref[...] = jnp.maximum(o, 0.0) if relu else o

    return pl.pallas_call(
        body,
        grid=(_tc_grid(N),),
        in_specs=[pl.BlockSpec((_BN, K), lambda i: (i, 0)),
                  pl.BlockSpec((K, M), lambda i: (0, 0)),
                  pl.BlockSpec((1, M), lambda i: (0, 0))],
        out_specs=pl.BlockSpec((_BN, M), lambda i: (i, 0)),
        out_shape=jax.ShapeDtypeStruct((N, M), jnp.float32),
        name=f"mm_{N}_{K}_{M}")


def _proj(x, w, b, relu=False):
    return _tc_mm(x.shape[0], x.shape[1], w.shape[1], relu)(x, w, b.reshape(1, -1))


@functools.cache
def _tc_cnn(N):
    def body(x_ref, k_ref, b_ref, s_ref, o_ref):
        h = x_ref[...]
        for i in range(3):
            h = jnp.dot(h, k_ref[i], preferred_element_type=jnp.float32)
            h = jnp.maximum(h + b_ref[i], 0.0)
        pool = jnp.dot(h, s_ref[0], preferred_element_type=jnp.float32)
        for k in range(1, 4):
            pool = jnp.maximum(
                pool, jnp.dot(h, s_ref[k], preferred_element_type=jnp.float32))
        o_ref[...] = pool

    return pl.pallas_call(
        body,
        grid=(_tc_grid(N),),
        in_specs=[pl.BlockSpec((_BN, 125), lambda i: (i, 0)),
                  pl.BlockSpec((3, 125, 125), lambda i: (0, 0, 0)),
                  pl.BlockSpec((3, 1, 125), lambda i: (0, 0, 0)),
                  pl.BlockSpec((4, 125, 45), lambda i: (0, 0, 0))],
        out_specs=pl.BlockSpec((_BN, 45), lambda i: (i, 0)),
        out_shape=jax.ShapeDtypeStruct((N, 45), jnp.float32),
        name="cnn")


@functools.cache
def _pool_sel():
    S = np.zeros((4, 125, 45), np.float32)
    win = [(0,), (1, 2), (3, 4)]
    for c in range(5):
        for i in range(3):
            for j in range(3):
                co = c * 9 + i * 3 + j
                R, C = win[i], win[j]
                for k, (a, b) in enumerate(((0, 0), (0, 1), (1, 0), (1, 1))):
                    r = R[min(a, len(R) - 1)]
                    cc = C[min(b, len(C) - 1)]
                    S[k, c * 25 + r * 5 + cc, co] = 1.0
    return S


def _cnn(x, p):
    mats, biases = [], []
    for i in (1, 2, 3):
        w = p['conv%d_w' % i]
        g, b2, m, v = (p['bn%d_g' % i], p['bn%d_b' % i],
                       p['bn%d_m' % i], p['bn%d_v' % i])
        scale = g / jnp.sqrt(v + 1e-5)
        shift = b2 - m * scale
        PI, PO, CC, DI, DJ = [], [], [], [], []
        for c in range(5):
            for ri in range(5):
                for cj in range(5):
                    for di in range(3):
                        for dj in range(3):
                            ii, jj = ri - di + 1, cj - dj + 1
                            if 0 <= ii < 5 and 0 <= jj < 5:
                                PI.append(c * 25 + ri * 5 + cj)
                                PO.append(c * 25 + ii * 5 + jj)
                                CC.append(c)
                                DI.append(di)
                                DJ.append(dj)
        K = jnp.zeros((125, 125), jnp.float32).at[
            np.array(PI), np.array(PO)].set(w[np.array(CC), 0, np.array(DI), np.array(DJ)])
        colscale = jnp.repeat(scale, 25)
        mats.append(K * colscale[None, :])
        biases.append((jnp.repeat(p['conv%d_b' % i], 25) * colscale
                       + jnp.repeat(shift, 25)).reshape(1, 125))
    kmat = jnp.stack(mats)
    bmat = jnp.stack(biases)
    S = jnp.asarray(_pool_sel())
    x2 = x.reshape(x.shape[0], 125)
    return _tc_cnn(x2.shape[0])(x2, kmat, bmat, S)


@functools.cache
def _tc_finish(N, H, relu, bn):
    F = H * 64

    def body(*refs):
        s_refs = refs[:H]
        d0_ref, d1_ref, bias_ref, sc_ref, sh_ref, o_ref = refs[H:]
        cnt = jnp.maximum(d0_ref[:, 2:3] + d1_ref[:, 2:3], 1.0)
        cols = []
        for h in range(H):
            den = d0_ref[:, h:h + 1] + d1_ref[:, h:h + 1]
            cols.append(s_refs[h][...] / (den * cnt + 1e-30))
        o = jnp.concatenate(cols, axis=1) + bias_ref[...]
        if bn:
            o = o * sc_ref[...] + sh_ref[...]
        o_ref[...] = jnp.maximum(o, 0.0) if relu else o

    return pl.pallas_call(
        body,
        grid=(_tc_grid(N),),
        in_specs=[pl.BlockSpec((_BN, 64), lambda i: (i, 0)) for _ in range(H)]
        + [pl.BlockSpec((_BN, 8), lambda i: (i, 0)),
           pl.BlockSpec((_BN, 8), lambda i: (i, 0)),
           pl.BlockSpec((1, F), lambda i: (0, 0)),
           pl.BlockSpec((1, F), lambda i: (0, 0)),
           pl.BlockSpec((1, F), lambda i: (0, 0))],
        out_specs=pl.BlockSpec((_BN, F), lambda i: (i, 0)),
        out_shape=jax.ShapeDtypeStruct((N, F), jnp.float32),
        name=f"finish_{N}_{H}")


@functools.cache
def _tc_mlp(N):
    def body(x_ref, w1, b1, w2, b2, w3, b3, o_ref):
        h = jnp.maximum(jnp.dot(x_ref[...], w1[...],
                                preferred_element_type=jnp.float32) + b1[...], 0.0)
        h = jnp.maximum(jnp.dot(h, w2[...],
                                preferred_element_type=jnp.float32) + b2[...], 0.0)
        o_ref[...] = jnp.dot(h, w3[...],
                             preferred_element_type=jnp.float32) + b3[...]

    return pl.pallas_call(
        body,
        grid=(_tc_grid(N),),
        in_specs=[pl.BlockSpec((_BN, 64), lambda i: (i, 0)),
                  pl.BlockSpec((64, 64), lambda i: (0, 0)),
                  pl.BlockSpec((1, 64), lambda i: (0, 0)),
                  pl.BlockSpec((64, 32), lambda i: (0, 0)),
                  pl.BlockSpec((1, 32), lambda i: (0, 0)),
                  pl.BlockSpec((32, 1), lambda i: (0, 0)),
                  pl.BlockSpec((1, 1), lambda i: (0, 0))],
        out_specs=pl.BlockSpec((_BN, 1), lambda i: (i, 0)),
        out_shape=jax.ShapeDtypeStruct((N, 1), jnp.float32),
        name="mlp")


def _pad1(a, n):
    return jnp.concatenate([a, jnp.zeros((n - a.shape[0],), a.dtype)])


def _sc_gat(x_src, x_dst, ei, p, name, heads, num_dst, bn=None, relu=False):
    if x_src is x_dst:
        w2 = jnp.concatenate([p[name + '_Wl'], p[name + '_Wr']], axis=1)
        b2 = jnp.concatenate([p[name + '_bl'], p[name + '_br']])
        xlr = _proj(x_src, w2, b2)
        F = heads * 64
        xl, xr = xlr[:, :F], xlr[:, F:]
    else:
        xl = _proj(x_src, p[name + '_Wl'], p[name + '_bl'])
        xr = _proj(x_dst, p[name + '_Wr'], p[name + '_br'])
    F = heads * 64
    E = ei.shape[1]
    E_pad = _ceil_to(E, _NW * _B)
    N_pad = _ceil_to(num_dst, 4 * _NS * 64)
    srcp = _pad1(ei[0], E_pad)
    dstp = _pad1(ei[1], E_pad)
    att = p[name + '_att'].reshape(-1)
    res = _phase_a(E, E_pad, F, heads, N_pad)(xl, xr, att, srcp, dstp)
    den2, msgs = res[0], res[1:]
    den2 = den2.reshape(_NC, N_pad, 8)
    if bn is not None:
        g, b2_, m, v = bn
        scale = g / jnp.sqrt(v + 1e-5)
        shift = b2_ - m * scale
    else:
        scale = jnp.ones((F,), jnp.float32)
        shift = jnp.zeros((F,), jnp.float32)
    outs = _phase_b(E_pad, heads, N_pad)(dstp, *msgs)
    out = _tc_finish(N_pad, heads, relu, bn is not None)(
        *outs, den2[0], den2[1], p[name + '_bias'].reshape(1, -1),
        scale.reshape(1, -1), shift.reshape(1, -1))
    return out[:num_dst]


def kernel(x_low, x_9x, x_25x, x_high, z_std_high, ei_low_9x, ei_9x_25x, ei_25x_high, ei_high, params):
    p = params
    h = _cnn(x_low, p)
    h = _sc_gat(h, x_9x, ei_low_9x, p, 'd1', 1, x_9x.shape[0])
    h = _sc_gat(h, x_25x, ei_9x_25x, p, 'd2', 1, x_25x.shape[0])
    h = _sc_gat(h, x_high, ei_25x_high, p, 'd3', 1, x_high.shape[0])
    h = jnp.concatenate([z_std_high, h], axis=-1)
    n = h.shape[0]
    loops = jnp.arange(n, dtype=ei_high.dtype)
    ei = jnp.concatenate([ei_high, jnp.stack([loops, loops])], axis=1)
    for i in (1, 2, 3, 4):
        bn = (p['pbn%d_g' % i], p['pbn%d_b' % i], p['pbn%d_m' % i], p['pbn%d_v' % i])
        h = _sc_gat(h, h, ei, p, 'p%d' % i, 2, n, bn=bn, relu=True)
    h = _sc_gat(h, h, ei, p, 'p5', 1, n, relu=True)
    return _tc_mlp(n)(h, p['fc1_w'], p['fc1_b'].reshape(1, -1),
                      p['fc2_w'], p['fc2_b'].reshape(1, -1),
                      p['fc3_w'], p['fc3_b'].reshape(1, -1))


# speedup vs baseline: 2.5265x; 1.2569x over previous
import functools

import numpy as np

import jax
import jax.numpy as jnp
from jax import lax
from jax.experimental import pallas as pl
from jax.experimental.pallas import tpu as pltpu
from jax.experimental.pallas import tpu_sc as plsc

_B = 128
_NC = 2
_NS = 16
_NW = _NC * _NS
_CP = pltpu.CompilerParams(use_tc_tiling_on_sc=False, needs_layout_passes=False)


def _iota16():
    return lax.iota(jnp.int32, 16)


def _splat_i(x):
    return jnp.full((16,), x, jnp.int32)


@functools.cache
def _phase_a(E_real, E_pad, F, H, N_pad):
    mesh = plsc.VectorSubcoreMesh(core_axis_name="c", subcore_axis_name="s",
                                  num_cores=_NC, num_subcores=_NS)
    nblk = E_pad // (_NW * _B)
    drpt = N_pad // _NS
    out_type = tuple([jax.ShapeDtypeStruct((_NC * N_pad, 8), jnp.float32)] +
                     [jax.ShapeDtypeStruct((E_pad, 64), jnp.float32)
                      for _ in range(H)])
    scratch = ([
        pltpu.VMEM((_B,), jnp.int32),
        pltpu.VMEM((_B,), jnp.int32),
        pltpu.VMEM((_B, F), jnp.float32),
        pltpu.VMEM((_B, F), jnp.float32),
        pltpu.VMEM((_B, 8), jnp.float32),
        pltpu.VMEM((F,), jnp.float32),
        pltpu.VMEM((_B, 8), jnp.float32),
        pltpu.VMEM_SHARED((N_pad, 8), jnp.float32),
    ] + [pltpu.VMEM((_B, 64), jnp.float32) for _ in range(H)]
      + [pltpu.VMEM((_B,), jnp.float32) for _ in range(H)]
      + [pltpu.SemaphoreType.DMA, pltpu.SemaphoreType.DMA])

    def body(*refs):
        (xl, xr, att, srcp, dstp, den2_o) = refs[:6]
        msg_o = refs[6:6 + H]
        (srcv, dstv, rows_l, rows_r, denblk, attv, zbuf, dacc) = refs[6 + H:14 + H]
        msgb = refs[14 + H:14 + 2 * H]
        exb = refs[14 + 2 * H:14 + 3 * H]
        sem1, sem2 = refs[14 + 3 * H:]
        c = lax.axis_index("c")
        s = lax.axis_index("s")
        wid = s * _NC + c
        it = _iota16()
        zf = jnp.zeros((16,), jnp.float32)
        for col in range(3):
            for r in range(_B // 16):
                plsc.store_scatter(zbuf, [r * 16 + it, _splat_i(col)], zf)
                plsc.store_scatter(denblk, [r * 16 + it, _splat_i(col)], zf)
        def zden(k, _):
            pltpu.sync_copy(zbuf, dacc.at[pl.ds(s * drpt + k * _B, _B)])
            return 0
        lax.fori_loop(0, drpt // _B, zden, 0)
        pltpu.sync_copy(att, attv)
        plsc.subcore_barrier()

        def block(i, _):
            base = (wid * nblk + i) * _B
            pltpu.sync_copy(srcp.at[pl.ds(base, _B)], srcv)
            pltpu.sync_copy(dstp.at[pl.ds(base, _B)], dstv)
            cp1 = pltpu.async_copy(xl.at[srcv], rows_l, sem1)
            cp2 = pltpu.async_copy(xr.at[dstv], rows_r, sem2)
            att_k = [attv[pl.ds(kk * 16, 16)] for kk in range(F // 16)]
            cp1.wait()
            cp2.wait()

            @plsc.parallel_loop(0, _B, unroll=2)
            def _edge_loop(j):
                for h in range(H):
                    acc = zf
                    vls = []
                    for kk in range(4):
                        col = h * 64 + kk * 16
                        vl = rows_l[j, pl.ds(col, 16)]
                        vr = rows_r[j, pl.ds(col, 16)]
                        vls.append(vl)
                        sm = vl + vr
                        e = jnp.maximum(sm, 0.2 * sm)
                        acc = acc + e * att_k[h * 4 + kk]
                    sc = jnp.sum(acc)
                    m16 = jnp.full((16,), base + j < E_real)
                    ex = jnp.where(
                        m16, jnp.exp(jnp.full((16,), sc, jnp.float32)), zf)
                    plsc.store_scatter(exb[h], [_splat_i(j)], ex)
                    for kk in range(4):
                        msgb[h][j, pl.ds(kk * 16, 16)] = vls[kk] * ex
            for g in range(_B // 16):
                rowi = g * 16 + it
                eids = base + rowi
                cntv = jnp.where(eids < E_real, 1.0, 0.0)
                plsc.store_scatter(denblk, [rowi, _splat_i(2)], cntv)
                for h in range(H):
                    ex16 = exb[h][pl.ds(g * 16, 16)]
                    plsc.store_scatter(denblk, [rowi, _splat_i(h)], ex16)
            for h in range(H):
                pltpu.sync_copy(msgb[h], msg_o[h].at[pl.ds(base, _B)])
            pltpu.sync_copy(denblk, dacc.at[dstv], add=True)
            return 0

        lax.fori_loop(0, nblk, block, 0)
        plsc.subcore_barrier()

        def wout(k, _):
            off = s * drpt + k * _B
            pltpu.sync_copy(dacc.at[pl.ds(off, _B)],
                            den2_o.at[pl.ds(c * N_pad + off, _B)])
            return 0
        lax.fori_loop(0, drpt // _B, wout, 0)

    return pl.kernel(body, out_type=out_type, mesh=mesh, compiler_params=_CP,
                     scratch_types=scratch, name=f"gat_a_{E_pad}_{F}_{H}")


@functools.cache
def _phase_b(E_pad, H, N_pad):
    mesh = plsc.VectorSubcoreMesh(core_axis_name="c", subcore_axis_name="s",
                                  num_cores=_NC, num_subcores=_NS)
    qsz = N_pad // 4
    arpt = qsz // _NS
    nbt = E_pad // (_NS * _B)
    out_type = tuple(jax.ShapeDtypeStruct((N_pad, 64), jnp.float32)
                     for _ in range(H))
    scratch = [
        pltpu.VMEM((_B,), jnp.int32),
        pltpu.VMEM((_B,), jnp.int32),
        pltpu.VMEM((_B,), jnp.int32),
        pltpu.VMEM((_B, 64), jnp.float32),
        pltpu.VMEM((_B, 64), jnp.float32),
        pltpu.VMEM((64, 64), jnp.float32),
        pltpu.VMEM_SHARED((qsz + 8, 64), jnp.float32),
        pltpu.SemaphoreType.DMA,
        pltpu.SemaphoreType.DMA,
        pltpu.SemaphoreType.DMA,
        pltpu.SemaphoreType.DMA,
    ]

    def body(*refs):
        dstp = refs[0]
        msgs = refs[1:1 + H]
        outs = refs[1 + H:1 + 2 * H]
        (dstv, idxb0, idxb1, rows0, rows1, zbuf, acc,
         sem0, sem1, semS0, semS1) = refs[1 + 2 * H:]
        c = lax.axis_index("c")
        s = lax.axis_index("s")
        it = _iota16()
        zf = jnp.zeros((16,), jnp.float32)

        def zz(i, _):
            fl = i * 16 + it
            plsc.store_scatter(zbuf, [fl // 64, fl % 64], zf)
            return 0
        lax.fori_loop(0, 64 * 64 // 16, zz, 0)

        def zacc(k, _):
            pltpu.sync_copy(zbuf, acc.at[pl.ds(s * arpt + k * 64, 64)])
            return 0

        for h in range(H):
            for qq in range(2):
                q = c * 2 + qq
                qbase = q * qsz
                lax.fori_loop(0, arpt // 64, zacc, 0)
                plsc.subcore_barrier()

                def _mkidx(qb, ib):
                    for g in range(_B // 16):
                        d16 = dstv[pl.ds(g * 16, 16)]
                        local = d16 - qb
                        msk = (local >= 0) & (local < qsz)
                        ib[pl.ds(g * 16, 16)] = jnp.where(msk, local, qsz)

                nh = nbt // 2
                tbase = s * nbt * _B
                pltpu.async_copy(msgs[h].at[pl.ds(tbase, _B)], rows0, sem0)

                def pair(i, _):
                    base0 = tbase + 2 * i * _B
                    base1 = base0 + _B

                    @pl.when(i > 0)
                    def _():
                        pltpu.make_async_copy(rows1, acc.at[idxb1],
                                              semS1).wait()
                    pltpu.async_copy(msgs[h].at[pl.ds(base1, _B)], rows1, sem1)

                    pltpu.sync_copy(dstp.at[pl.ds(base0, _B)], dstv)
                    _mkidx(qbase, idxb0)
                    pltpu.make_async_copy(msgs[h].at[pl.ds(base0, _B)],
                                          rows0, sem0).wait()
                    pltpu.async_copy(rows0, acc.at[idxb0], semS0, add=True)

                    pltpu.sync_copy(dstp.at[pl.ds(base1, _B)], dstv)
                    _mkidx(qbase, idxb1)
                    pltpu.make_async_copy(msgs[h].at[pl.ds(base1, _B)],
                                          rows1, sem1).wait()
                    pltpu.async_copy(rows1, acc.at[idxb1], semS1, add=True)

                    @pl.when(i + 1 < nh)
                    def _():
                        pltpu.make_async_copy(rows0, acc.at[idxb0],
                                              semS0).wait()
                        pltpu.async_copy(msgs[h].at[pl.ds(base0 + 2 * _B, _B)],
                                         rows0, sem0)
                    return 0

                lax.fori_loop(0, nh, pair, 0)
                pltpu.make_async_copy(rows0, acc.at[idxb0], semS0).wait()
                pltpu.make_async_copy(rows1, acc.at[idxb1], semS1).wait()
                plsc.subcore_barrier()

                def wout(k, _):
                    off = s * arpt + k * 64
                    pltpu.sync_copy(acc.at[pl.ds(off, 64)],
                                    outs[h].at[pl.ds(qbase + off, 64)])
                    return 0
                lax.fori_loop(0, arpt // 64, wout, 0)
                if h + 1 < H or qq == 0:
                    plsc.subcore_barrier()

    return pl.kernel(body, out_type=out_type, mesh=mesh, compiler_params=_CP,
                     scratch_types=scratch, name=f"gat_b_{E_pad}_{H}")


def _ceil_to(x, m):
    return (x + m - 1) // m * m


_BN = 1024


def _tc_grid(n):
    return (n + _BN - 1) // _BN


@functools.cache
def _tc_mm(N, K, M, relu):
    def body(x_ref, w_ref, b_ref, o_ref):
        o = jnp.dot(x_ref[...], w_ref[...],
                    preferred_element_type=jnp.float32) + b_ref[...]
        o_ref[...] = jnp.maximum(o, 0.0) if relu else o

    return pl.pallas_call(
        body,
        grid=(_tc_grid(N),),
        in_specs=[pl.BlockSpec((_BN, K), lambda i: (i, 0)),
                  pl.BlockSpec((K, M), lambda i: (0, 0)),
                  pl.BlockSpec((1, M), lambda i: (0, 0))],
        out_specs=pl.BlockSpec((_BN, M), lambda i: (i, 0)),
        out_shape=jax.ShapeDtypeStruct((N, M), jnp.float32),
        name=f"mm_{N}_{K}_{M}")


def _proj(x, w, b, relu=False):
    return _tc_mm(x.shape[0], x.shape[1], w.shape[1], relu)(x, w, b.reshape(1, -1))


@functools.cache
def _tc_cnn(N):
    def body(x_ref, k_ref, b_ref, s_ref, o_ref):
        h = x_ref[...]
        for i in range(3):
            h = jnp.dot(h, k_ref[i], preferred_element_type=jnp.float32)
            h = jnp.maximum(h + b_ref[i], 0.0)
        pool = jnp.dot(h, s_ref[0], preferred_element_type=jnp.float32)
        for k in range(1, 4):
            pool = jnp.maximum(
                pool, jnp.dot(h, s_ref[k], preferred_element_type=jnp.float32))
        o_ref[...] = pool

    return pl.pallas_call(
        body,
        grid=(_tc_grid(N),),
        in_specs=[pl.BlockSpec((_BN, 125), lambda i: (i, 0)),
                  pl.BlockSpec((3, 125, 125), lambda i: (0, 0, 0)),
                  pl.BlockSpec((3, 1, 125), lambda i: (0, 0, 0)),
                  pl.BlockSpec((4, 125, 45), lambda i: (0, 0, 0))],
        out_specs=pl.BlockSpec((_BN, 45), lambda i: (i, 0)),
        out_shape=jax.ShapeDtypeStruct((N, 45), jnp.float32),
        name="cnn")


@functools.cache
def _pool_sel():
    S = np.zeros((4, 125, 45), np.float32)
    win = [(0,), (1, 2), (3, 4)]
    for c in range(5):
        for i in range(3):
            for j in range(3):
                co = c * 9 + i * 3 + j
                R, C = win[i], win[j]
                for k, (a, b) in enumerate(((0, 0), (0, 1), (1, 0), (1, 1))):
                    r = R[min(a, len(R) - 1)]
                    cc = C[min(b, len(C) - 1)]
                    S[k, c * 25 + r * 5 + cc, co] = 1.0
    return S


def _cnn(x, p):
    mats, biases = [], []
    for i in (1, 2, 3):
        w = p['conv%d_w' % i]
        g, b2, m, v = (p['bn%d_g' % i], p['bn%d_b' % i],
                       p['bn%d_m' % i], p['bn%d_v' % i])
        scale = g / jnp.sqrt(v + 1e-5)
        shift = b2 - m * scale
        PI, PO, CC, DI, DJ = [], [], [], [], []
        for c in range(5):
            for ri in range(5):
                for cj in range(5):
                    for di in range(3):
                        for dj in range(3):
                            ii, jj = ri - di + 1, cj - dj + 1
                            if 0 <= ii < 5 and 0 <= jj < 5:
                                PI.append(c * 25 + ri * 5 + cj)
                                PO.append(c * 25 + ii * 5 + jj)
                                CC.append(c)
                                DI.append(di)
                                DJ.append(dj)
        K = jnp.zeros((125, 125), jnp.float32).at[
            np.array(PI), np.array(PO)].set(w[np.array(CC), 0, np.array(DI), np.array(DJ)])
        colscale = jnp.repeat(scale, 25)
        mats.append(K * colscale[None, :])
        biases.append((jnp.repeat(p['conv%d_b' % i], 25) * colscale
                       + jnp.repeat(shift, 25)).reshape(1, 125))
    kmat = jnp.stack(mats)
    bmat = jnp.stack(biases)
    S = jnp.asarray(_pool_sel())
    x2 = x.reshape(x.shape[0], 125)
    return _tc_cnn(x2.shape[0])(x2, kmat, bmat, S)


@functools.cache
def _tc_finish(N, H, relu, bn):
    F = H * 64

    def body(*refs):
        s_refs = refs[:H]
        d0_ref, d1_ref, bias_ref, sc_ref, sh_ref, o_ref = refs[H:]
        cnt = jnp.maximum(d0_ref[:, 2:3] + d1_ref[:, 2:3], 1.0)
        cols = []
        for h in range(H):
            den = d0_ref[:, h:h + 1] + d1_ref[:, h:h + 1]
            cols.append(s_refs[h][...] / (den * cnt + 1e-30))
        o = jnp.concatenate(cols, axis=1) + bias_ref[...]
        if bn:
            o = o * sc_ref[...] + sh_ref[...]
        o_ref[...] = jnp.maximum(o, 0.0) if relu else o

    return pl.pallas_call(
        body,
        grid=(_tc_grid(N),),
        in_specs=[pl.BlockSpec((_BN, 64), lambda i: (i, 0)) for _ in range(H)]
        + [pl.BlockSpec((_BN, 8), lambda i: (i, 0)),
           pl.BlockSpec((_BN, 8), lambda i: (i, 0)),
           pl.BlockSpec((1, F), lambda i: (0, 0)),
           pl.BlockSpec((1, F), lambda i: (0, 0)),
           pl.BlockSpec((1, F), lambda i: (0, 0))],
        out_specs=pl.BlockSpec((_BN, F), lambda i: (i, 0)),
        out_shape=jax.ShapeDtypeStruct((N, F), jnp.float32),
        name=f"finish_{N}_{H}")


@functools.cache
def _tc_mlp(N):
    def body(x_ref, w1, b1, w2, b2, w3, b3, o_ref):
        h = jnp.maximum(jnp.dot(x_ref[...], w1[...],
                                preferred_element_type=jnp.float32) + b1[...], 0.0)
        h = jnp.maximum(jnp.dot(h, w2[...],
                                preferred_element_type=jnp.float32) + b2[...], 0.0)
        o_ref[...] = jnp.dot(h, w3[...],
                             preferred_element_type=jnp.float32) + b3[...]

    return pl.pallas_call(
        body,
        grid=(_tc_grid(N),),
        in_specs=[pl.BlockSpec((_BN, 64), lambda i: (i, 0)),
                  pl.BlockSpec((64, 64), lambda i: (0, 0)),
                  pl.BlockSpec((1, 64), lambda i: (0, 0)),
                  pl.BlockSpec((64, 32), lambda i: (0, 0)),
                  pl.BlockSpec((1, 32), lambda i: (0, 0)),
                  pl.BlockSpec((32, 1), lambda i: (0, 0)),
                  pl.BlockSpec((1, 1), lambda i: (0, 0))],
        out_specs=pl.BlockSpec((_BN, 1), lambda i: (i, 0)),
        out_shape=jax.ShapeDtypeStruct((N, 1), jnp.float32),
        name="mlp")


def _pad1(a, n):
    return jnp.concatenate([a, jnp.zeros((n - a.shape[0],), a.dtype)])


def _sc_gat(x_src, x_dst, ei, p, name, heads, num_dst, bn=None, relu=False):
    if x_src is x_dst:
        w2 = jnp.concatenate([p[name + '_Wl'], p[name + '_Wr']], axis=1)
        b2 = jnp.concatenate([p[name + '_bl'], p[name + '_br']])
        xlr = _proj(x_src, w2, b2)
        F = heads * 64
        xl, xr = xlr[:, :F], xlr[:, F:]
    else:
        xl = _proj(x_src, p[name + '_Wl'], p[name + '_bl'])
        xr = _proj(x_dst, p[name + '_Wr'], p[name + '_br'])
    F = heads * 64
    E = ei.shape[1]
    E_pad = _ceil_to(E, _NW * _B)
    N_pad = _ceil_to(num_dst, 4 * _NS * 64)
    srcp = _pad1(ei[0], E_pad)
    dstp = _pad1(ei[1], E_pad)
    att = p[name + '_att'].reshape(-1)
    res = _phase_a(E, E_pad, F, heads, N_pad)(xl, xr, att, srcp, dstp)
    den2, msgs = res[0], res[1:]
    den2 = den2.reshape(_NC, N_pad, 8)
    if bn is not None:
        g, b2_, m, v = bn
        scale = g / jnp.sqrt(v + 1e-5)
        shift = b2_ - m * scale
    else:
        scale = jnp.ones((F,), jnp.float32)
        shift = jnp.zeros((F,), jnp.float32)
    outs = _phase_b(E_pad, heads, N_pad)(dstp, *msgs)
    out = _tc_finish(N_pad, heads, relu, bn is not None)(
        *outs, den2[0], den2[1], p[name + '_bias'].reshape(1, -1),
        scale.reshape(1, -1), shift.reshape(1, -1))
    return out[:num_dst]


def kernel(x_low, x_9x, x_25x, x_high, z_std_high, ei_low_9x, ei_9x_25x, ei_25x_high, ei_high, params):
    p = params
    h = _cnn(x_low, p)
    h = _sc_gat(h, x_9x, ei_low_9x, p, 'd1', 1, x_9x.shape[0])
    h = _sc_gat(h, x_25x, ei_9x_25x, p, 'd2', 1, x_25x.shape[0])
    h = _sc_gat(h, x_high, ei_25x_high, p, 'd3', 1, x_high.shape[0])
    h = jnp.concatenate([z_std_high, h], axis=-1)
    n = h.shape[0]
    loops = jnp.arange(n, dtype=ei_high.dtype)
    ei = jnp.concatenate([ei_high, jnp.stack([loops, loops])], axis=1)
    for i in (1, 2, 3, 4):
        bn = (p['pbn%d_g' % i], p['pbn%d_b' % i], p['pbn%d_m' % i], p['pbn%d_v' % i])
        h = _sc_gat(h, h, ei, p, 'p%d' % i, 2, n, bn=bn, relu=True)
    h = _sc_gat(h, h, ei, p, 'p5', 1, n, relu=True)
    return _tc_mlp(n)(h, p['fc1_w'], p['fc1_b'].reshape(1, -1),
                      p['fc2_w'], p['fc2_b'].reshape(1, -1),
                      p['fc3_w'], p['fc3_b'].reshape(1, -1))
